# Initial kernel scaffold; baseline (speedup 1.0000x reference)
import jax
import jax.numpy as jnp
from jax.experimental import pallas as pl


def kernel(x, edge_index, batch, fc_W, c0_W, c0_asrc, c0_adst, c0_b, c1_W, c1_asrc, c1_adst, c1_b, lp0_W, lp0_b, lp1_W, lp1_b, lp2_W, lp2_b, bn0_g, bn0_b, bn1_g, bn1_b, bn2_g, bn2_b, lin_W, lin_b):
    raise NotImplementedError("write your pallas kernel here")


# speedup vs baseline: 3.4533x; 3.4533x over previous
import functools

import jax
import jax.numpy as jnp
from jax import lax
from jax.experimental import pallas as pl
from jax.experimental.pallas import tpu as pltpu
from jax.experimental.pallas import tpu_sc as plsc

N_NODES = 10000
N_EDGES = 320000
NFEAT = 128
NHID = 128
NCLASS = 10
NUM_GRAPHS = 64

NT = 32
CPT = NHID // NT
CH = 2000
NCHUNK = N_EDGES // CH
GPC = CH // 16


def _pre_body(x_ref, fcW_ref, W_ref, asrc_ref, adst_ref,
              hid_t_ref, hw_t_ref, s_ref, t_ref):
    hid_t = lax.dot_general(fcW_ref[...], x_ref[...],
                            (((0,), (1,)), ((), ())),
                            preferred_element_type=jnp.float32)
    hid_t_ref[...] = hid_t
    hw_t = lax.dot_general(W_ref[...], hid_t,
                           (((0,), (0,)), ((), ())),
                           preferred_element_type=jnp.float32)
    hw_t_ref[...] = hw_t
    s_ref[...] = lax.dot_general(asrc_ref[...], hw_t,
                                 (((1,), (0,)), ((), ())),
                                 preferred_element_type=jnp.float32)
    t_ref[...] = lax.dot_general(adst_ref[...], hw_t,
                                 (((1,), (0,)), ((), ())),
                                 preferred_element_type=jnp.float32)


def _mid_body(out0_ref, b0_ref, W1_ref, asrc_ref, adst_ref,
              h1_t_ref, hw_t_ref, s_ref, t_ref):
    v = out0_ref[...] + b0_ref[...]
    h1 = jnp.where(v > 0.0, v, jnp.exp(jnp.minimum(v, 0.0)) - 1.0)
    h1_t_ref[...] = h1
    hw_t = lax.dot_general(W1_ref[...], h1,
                           (((0,), (0,)), ((), ())),
                           preferred_element_type=jnp.float32)
    hw_t_ref[...] = hw_t
    s_ref[...] = lax.dot_general(asrc_ref[...], hw_t,
                                 (((1,), (0,)), ((), ())),
                                 preferred_element_type=jnp.float32)
    t_ref[...] = lax.dot_general(adst_ref[...], hw_t,
                                 (((1,), (0,)), ((), ())),
                                 preferred_element_type=jnp.float32)


def _bn(h, g, b):
    mu = jnp.mean(h, axis=0, keepdims=True)
    var = jnp.mean((h - mu) * (h - mu), axis=0, keepdims=True)
    return (h - mu) * lax.rsqrt(var + 1e-5) * g + b


def _readout_body(batch_ref, x_ref, h1_ref, out1_ref, b1_ref,
                  bn0g_ref, bn0b_ref, bn1g_ref, bn1b_ref, bn2g_ref, bn2b_ref,
                  lp0W_ref, lp0b_ref, lp1W_ref, lp1b_ref, lp2W_ref, lp2b_ref,
                  linW_ref, linb_ref, out_ref):
    oh = (lax.broadcasted_iota(jnp.int32, (NUM_GRAPHS, N_NODES), 0)
          == batch_ref[...]).astype(jnp.float32)
    cnt = jnp.maximum(jnp.sum(oh, axis=1, keepdims=True), 1.0)
    p0 = jnp.dot(oh, x_ref[...], preferred_element_type=jnp.float32) / cnt
    p1 = lax.dot_general(oh, h1_ref[...], (((1,), (1,)), ((), ())),
                         preferred_element_type=jnp.float32) / cnt
    h2 = out1_ref[...] + b1_ref[...]
    p2 = lax.dot_general(oh, h2, (((1,), (1,)), ((), ())),
                         preferred_element_type=jnp.float32) / cnt
    score = jnp.maximum(
        jnp.dot(_bn(p0, bn0g_ref[...], bn0b_ref[...]), lp0W_ref[...],
                preferred_element_type=jnp.float32) + lp0b_ref[...], 0.0)
    score = score + jnp.maximum(
        jnp.dot(_bn(p1, bn1g_ref[...], bn1b_ref[...]), lp1W_ref[...],
                preferred_element_type=jnp.float32) + lp1b_ref[...], 0.0)
    score = score + jnp.maximum(
        jnp.dot(_bn(p2, bn2g_ref[...], bn2b_ref[...]), lp2W_ref[...],
                preferred_element_type=jnp.float32) + lp2b_ref[...], 0.0)
    logits = jnp.dot(score, linW_ref[...],
                     preferred_element_type=jnp.float32) + linb_ref[...]
    m = jnp.max(logits, axis=1, keepdims=True)
    z = logits - m
    out_ref[...] = z - jnp.log(jnp.sum(jnp.exp(z), axis=1, keepdims=True))


_F32 = jnp.float32


def _pre_call(x, fc_W, W, asrc, adst):
    return pl.pallas_call(
        _pre_body,
        out_shape=[
            jax.ShapeDtypeStruct((NHID, N_NODES), _F32),
            jax.ShapeDtypeStruct((NHID, N_NODES), _F32),
            jax.ShapeDtypeStruct((1, N_NODES), _F32),
            jax.ShapeDtypeStruct((1, N_NODES), _F32),
        ],
    )(x, fc_W, W, asrc.reshape(1, NHID), adst.reshape(1, NHID))


def _mid_call(out0_t, b0, W1, asrc, adst):
    return pl.pallas_call(
        _mid_body,
        out_shape=[
            jax.ShapeDtypeStruct((NHID, N_NODES), _F32),
            jax.ShapeDtypeStruct((NHID, N_NODES), _F32),
            jax.ShapeDtypeStruct((1, N_NODES), _F32),
            jax.ShapeDtypeStruct((1, N_NODES), _F32),
        ],
    )(out0_t, b0.reshape(NHID, 1), W1, asrc.reshape(1, NHID),
      adst.reshape(1, NHID))


def _readout_call(batch, x, h1_t, out1_t, b1, bn0g, bn0b, bn1g, bn1b,
                  bn2g, bn2b, lp0W, lp0b, lp1W, lp1b, lp2W, lp2b,
                  linW, linb):
    return pl.pallas_call(
        _readout_body,
        out_shape=jax.ShapeDtypeStruct((NUM_GRAPHS, NCLASS), _F32),
    )(batch.reshape(1, N_NODES), x, h1_t, out1_t, b1.reshape(NHID, 1),
      bn0g.reshape(1, NFEAT), bn0b.reshape(1, NFEAT),
      bn1g.reshape(1, NHID), bn1b.reshape(1, NHID),
      bn2g.reshape(1, NHID), bn2b.reshape(1, NHID),
      lp0W, lp0b.reshape(1, NHID), lp1W, lp1b.reshape(1, NHID),
      lp2W, lp2b.reshape(1, NHID),
      linW, linb.reshape(1, NCLASS))


_MESH = plsc.VectorSubcoreMesh(core_axis_name="c", subcore_axis_name="s")


@functools.partial(
    pl.kernel,
    mesh=_MESH,
    compiler_params=pltpu.CompilerParams(
        use_tc_tiling_on_sc=False, needs_layout_passes=False),
    out_type=jax.ShapeDtypeStruct((NHID, N_NODES), _F32),
    scratch_types=[
        pltpu.VMEM((N_NODES,), _F32),
        pltpu.VMEM((N_NODES,), _F32),
        pltpu.VMEM((N_NODES,), _F32),
        pltpu.VMEM((CPT, N_NODES), _F32),
        pltpu.VMEM((CPT, N_NODES), _F32),
        pltpu.VMEM((CH,), jnp.int32),
        pltpu.VMEM((CH,), jnp.int32),
    ],
)
def _conv_edges(hw_t, s_hbm, t_hbm, src_hbm, dst_hbm, out_hbm,
                s_v, t_v, d_v, hw_v, o_v, src_v, dst_v):
    cid = lax.axis_index("c")
    sid = lax.axis_index("s")
    wid = sid * 2 + cid
    base = pl.multiple_of(wid * CPT, CPT)
    pltpu.sync_copy(s_hbm, s_v)
    pltpu.sync_copy(t_hbm, t_v)
    pltpu.sync_copy(hw_t.at[pl.ds(base, CPT)], hw_v)

    zeros16 = jnp.zeros((16,), _F32)

    def zero_body(i, _):
        off = pl.multiple_of(i * 16, 16)
        d_v[pl.ds(off, 16)] = zeros16
        for r in range(CPT):
            o_v[r, pl.ds(off, 16)] = zeros16
        return 0

    lax.fori_loop(0, N_NODES // 16, zero_body, 0)

    def edge_e(g):
        off = pl.multiple_of(g * 16, 16)
        i16s = src_v[pl.ds(off, 16)]
        i16d = dst_v[pl.ds(off, 16)]
        sv = plsc.load_gather(s_v, [i16s])
        tv = plsc.load_gather(t_v, [i16d])
        e = sv + tv
        e = jnp.where(e >= 0.0, e, 0.2 * e)
        return i16s, i16d, jnp.exp(e)

    def denom_chunk(ci, _):
        cbase = pl.multiple_of(ci * CH, 8)
        pltpu.sync_copy(src_hbm.at[pl.ds(cbase, CH)], src_v)
        pltpu.sync_copy(dst_hbm.at[pl.ds(cbase, CH)], dst_v)

        def g_body(g, _):
            _, i16d, ee = edge_e(g)
            plsc.addupdate_scatter(d_v, [i16d], ee)
            return 0

        lax.fori_loop(0, GPC, g_body, 0)
        return 0

    lax.fori_loop(0, NCHUNK, denom_chunk, 0)

    def out_chunk(ci, _):
        cbase = pl.multiple_of(ci * CH, 8)
        pltpu.sync_copy(src_hbm.at[pl.ds(cbase, CH)], src_v)
        pltpu.sync_copy(dst_hbm.at[pl.ds(cbase, CH)], dst_v)

        def g_body(g, _):
            i16s, i16d, ee = edge_e(g)
            dg = plsc.load_gather(d_v, [i16d])
            att = ee / (dg + 1e-16)
            for r in range(CPT):
                ridx = jnp.full((16,), r, jnp.int32)
                hv = plsc.load_gather(hw_v, [ridx, i16s])
                plsc.addupdate_scatter(o_v, [ridx, i16d], hv * att)
            return 0

        lax.fori_loop(0, GPC, g_body, 0)
        return 0

    lax.fori_loop(0, NCHUNK, out_chunk, 0)

    pltpu.sync_copy(o_v, out_hbm.at[pl.ds(base, CPT)])


def kernel(x, edge_index, batch, fc_W, c0_W, c0_asrc, c0_adst, c0_b,
           c1_W, c1_asrc, c1_adst, c1_b, lp0_W, lp0_b, lp1_W, lp1_b,
           lp2_W, lp2_b, bn0_g, bn0_b, bn1_g, bn1_b, bn2_g, bn2_b,
           lin_W, lin_b):
    src = edge_index[0]
    dst = edge_index[1]
    hid_t, hw0_t, s0, t0 = _pre_call(x, fc_W, c0_W, c0_asrc, c0_adst)
    out0_t = _conv_edges(hw0_t, s0.reshape(-1), t0.reshape(-1), src, dst)
    h1_t, hw1_t, s1, t1 = _mid_call(out0_t, c0_b, c1_W, c1_asrc, c1_adst)
    out1_t = _conv_edges(hw1_t, s1.reshape(-1), t1.reshape(-1), src, dst)
    return _readout_call(batch, x, h1_t, out1_t, c1_b,
                         bn0_g, bn0_b, bn1_g, bn1_b, bn2_g, bn2_b,
                         lp0_W, lp0_b, lp1_W, lp1_b, lp2_W, lp2_b,
                         lin_W, lin_b)
